# baseline (speedup 1.0000x reference)
import functools

import jax
import jax.numpy as jnp
from jax.experimental import pallas as pl

BASE_SPARSITY = 0.1


def _router_block(xt_ref, wct_ref, bias_ref, patt_ref, out_ref, misc_ref, *,
                  k, p):
    xt = xt_ref[...]
    T = xt.shape[1]
    logits_t = jnp.dot(wct_ref[...], xt, preferred_element_type=jnp.float32)
    logits_t = logits_t + bias_ref[...]
    sel_logits = logits_t[:p, :]
    m = jnp.max(sel_logits, axis=0, keepdims=True)
    e = jnp.exp(sel_logits - m)
    w = e / jnp.sum(e, axis=0, keepdims=True)
    intensity = jax.nn.sigmoid(logits_t[p:p + 1, :])

    g = jnp.dot(patt_ref[...], w, preferred_element_type=jnp.float32)
    g = g * intensity
    bits = jax.lax.bitcast_convert_type(g, jnp.int32) & jnp.int32(0x7FFFFFFF)

    def _colsum(a):
        r = a.shape[0]
        while r > 8:
            a = a.reshape(2, r // 2, a.shape[1]).sum(axis=0)
            r //= 2
        return a.sum(axis=0, keepdims=True)

    def body(_, carry):
        lo, hi = carry
        mid = lo + jax.lax.shift_right_logical(hi - lo, 1)
        lt = jax.lax.shift_right_logical(bits - mid, 31)
        cnt_lt = _colsum(lt)
        ge = cnt_lt <= bits.shape[0] - k
        return jnp.where(ge, mid, lo), jnp.where(ge, hi, mid)

    lo0 = jnp.zeros((1, T), jnp.int32)
    hi0 = jnp.full((1, T), jnp.int32(0x7F800001))
    lo, _ = jax.lax.fori_loop(0, 31, body, (lo0, hi0), unroll=4)

    out_t = jnp.where(bits >= lo, g, 0.0)
    out_ref[...] = out_t.T

    ent = -jnp.sum(w * jnp.log(w + 1e-08), axis=0, keepdims=True)
    misc_ref[...] = jnp.concatenate(
        [w, ent, intensity, jnp.zeros((6, T), jnp.float32)], axis=0)


@functools.partial(jax.jit, static_argnames=())
def kernel(x, flow_patterns, sel_w, sel_b, int_w, int_b):
    B, S, D_IN = x.shape
    P, O, I = flow_patterns.shape
    N = B * S
    OI = O * I
    base_k = max(1, int(OI * BASE_SPARSITY))
    k = min(base_k, OI // 2)

    xt = x.reshape(N, D_IN).T
    wct = jnp.zeros((16, D_IN), jnp.float32)
    wct = wct.at[:P, :].set(sel_w).at[P, :].set(int_w[0])
    bias = jnp.zeros((16, 1), jnp.float32)
    bias = bias.at[:P, 0].set(sel_b).at[P, 0].set(int_b[0])
    patt = flow_patterns.reshape(P, OI).T

    T = 256
    G = N // T
    out, misc = pl.pallas_call(
        functools.partial(_router_block, k=k, p=P),
        grid=(G,),
        in_specs=[
            pl.BlockSpec((D_IN, T), lambda i: (0, i)),
            pl.BlockSpec((16, D_IN), lambda i: (0, 0)),
            pl.BlockSpec((16, 1), lambda i: (0, 0)),
            pl.BlockSpec((OI, P), lambda i: (0, 0)),
        ],
        out_specs=[
            pl.BlockSpec((T, OI), lambda i: (i, 0)),
            pl.BlockSpec((16, T), lambda i: (0, i)),
        ],
        out_shape=[
            jax.ShapeDtypeStruct((N, OI), jnp.float32),
            jax.ShapeDtypeStruct((16, N), jnp.float32),
        ],
    )(xt, wct, bias, patt)

    pattern_entropy = jnp.mean(misc[P, :])
    flow_intensity_mean = jnp.mean(misc[P + 1, :])
    pattern_diversity = jnp.std(jnp.mean(misc[:P, :], axis=1), ddof=1)
    return (out.reshape(B, S, O, I), pattern_entropy, flow_intensity_mean,
            pattern_diversity)
